# baseline (speedup 1.0000x reference)
import jax
import jax.numpy as jnp
from jax.experimental import pallas as pl
from jax.experimental.pallas import tpu as pltpu

_K_TOP = 512
_EPS = 0.1
_MAX_ITER = 200
_ROWS = 128
_CHUNK = 256
_TOL = 2e-6


def _stk_kernel(s_ref, o_ref, q0_ref, q2_ref, af0_ref, af2_ref):
    rows = s_ref.shape[0]
    n = s_ref.shape[1]
    n_chunks = n // _CHUNK
    nf = jnp.float32(n)
    nu0n = jnp.float32(_K_TOP)
    nu1n = jnp.float32(n - 2 * _K_TOP)
    nu2n = jnp.float32(_K_TOP)
    one = jnp.float32(1.0)
    tol = jnp.float32(_TOL)

    def fold(p, op):
        t = p[:, 0:128]
        for k in range(1, _CHUNK // 128):
            t = op(t, p[:, k * 128:(k + 1) * 128])
        return t

    add = jnp.add

    first = s_ref[:, 0:_CHUNK]
    mn = fold(first, jnp.minimum)
    mx = fold(first, jnp.maximum)
    for c in range(1, n_chunks):
        sc = s_ref[:, c * _CHUNK:(c + 1) * _CHUNK]
        mn = jnp.minimum(mn, fold(sc, jnp.minimum))
        mx = jnp.maximum(mx, fold(sc, jnp.maximum))
    smin = jnp.min(mn, axis=1, keepdims=True)
    smax = jnp.max(mx, axis=1, keepdims=True)

    g = jnp.float32(1.0 / _EPS) / (smax - smin + jnp.float32(1e-12))
    u0 = jnp.float32(0.25 / _EPS) + smin * g
    u2 = jnp.float32(0.75 / _EPS) + smin * g

    acc1 = jnp.zeros((rows, 128), jnp.float32)
    acc2 = jnp.zeros((rows, 128), jnp.float32)
    for c in range(n_chunks):
        sl = slice(c * _CHUNK, (c + 1) * _CHUNK)
        t = s_ref[:, sl] * g
        q0c = jnp.exp(u0 - t)
        q2c = jnp.exp(t - u2)
        q0_ref[:, sl] = q0c
        q2_ref[:, sl] = q2c
        w = one / (one + q0c + q2c)
        acc1 = acc1 + fold(w, add)
        acc2 = acc2 + fold(q2c * w, add)
    r1 = jnp.sum(acc1, axis=1, keepdims=True)
    r2 = jnp.sum(acc2, axis=1, keepdims=True)

    def column_sums(a0, a1, a2):
        half = rows // 2
        r1h = []
        r2h = []
        for h in range(2):
            rsl = slice(h * half, (h + 1) * half)
            a0h = a0[rsl]
            a1h = a1[rsl]
            a2h = a2[rsl]
            acc1 = jnp.zeros((half, 128), jnp.float32)
            acc2 = jnp.zeros((half, 128), jnp.float32)
            for c in range(n_chunks):
                sl = slice(c * _CHUNK, (c + 1) * _CHUNK)
                q0c = q0_ref[rsl, sl]
                q2c = q2_ref[rsl, sl]
                w = one / (a1h + q0c * a0h + q2c * a2h)
                acc1 = acc1 + fold(w, add)
                acc2 = acc2 + fold(q2c * w, add)
            r1h.append(jnp.sum(acc1, axis=1, keepdims=True))
            r2h.append(jnp.sum(acc2, axis=1, keepdims=True))
        r1 = jnp.concatenate(r1h, axis=0)
        r2 = jnp.concatenate(r2h, axis=0)
        return r1, r2

    def new_scales(a0, a1, a2, r1, r2):
        a0n = nu0n * a0 / (nf - a1 * r1 - a2 * r2)
        return a0n, nu1n / r1, nu2n / r2

    a0, a1, a2 = new_scales(one, one, one, r1, r2)

    def cond(carry):
        t, _, _, _, delta = carry
        return jnp.logical_and(t < _MAX_ITER - 1, delta > 0)

    def body(carry):
        t, a0, a1, a2, _ = carry
        r1, r2 = column_sums(a0, a1, a2)
        a0n, a1n, a2n = new_scales(a0, a1, a2, r1, r2)
        d = jnp.maximum(jnp.abs(a0n - a0) - tol * a0,
                        jnp.abs(a1n - a1) - tol * a1)
        d = jnp.maximum(d, jnp.abs(a2n - a2) - tol * a2)
        return (t + 1, a0n, a1n, a2n, jnp.max(d))

    t, a0, a1, a2, delta = jax.lax.while_loop(
        cond, body, (1, a0, a1, a2, jnp.float32(1.0)))

    af0_ref[...] = a0
    af2_ref[...] = a2

    @pl.when(delta > 0)
    def _():
        r1f, r2f = column_sums(a0, a1, a2)
        a0n, _, a2n = new_scales(a0, a1, a2, r1f, r2f)
        af0_ref[...] = a0n
        af2_ref[...] = a2n

    a0n = af0_ref[...]
    a2n = af2_ref[...]
    for c in range(n_chunks):
        sl = slice(c * _CHUNK, (c + 1) * _CHUNK)
        q0c = q0_ref[:, sl]
        q2c = q2_ref[:, sl]
        w = one / (a1 + q0c * a0 + q2c * a2)
        o_ref[:, sl] = (q2c * a2n - q0c * a0n) * w


@jax.jit
def kernel(scores):
    b, n = scores.shape
    return pl.pallas_call(
        _stk_kernel,
        grid=(b // _ROWS,),
        in_specs=[pl.BlockSpec((_ROWS, n), lambda i: (i, 0))],
        out_specs=pl.BlockSpec((_ROWS, n), lambda i: (i, 0)),
        out_shape=jax.ShapeDtypeStruct((b, n), jnp.float32),
        scratch_shapes=[
            pltpu.VMEM((_ROWS, n), jnp.float32),
            pltpu.VMEM((_ROWS, n), jnp.float32),
            pltpu.VMEM((_ROWS, 1), jnp.float32),
            pltpu.VMEM((_ROWS, 1), jnp.float32),
        ],
        compiler_params=pltpu.CompilerParams(
            dimension_semantics=("parallel",),
        ),
    )(scores)


# speedup vs baseline: 1.1268x; 1.1268x over previous
import jax
import jax.numpy as jnp
from jax.experimental import pallas as pl
from jax.experimental.pallas import tpu as pltpu

_K_TOP = 512
_EPS = 0.1
_MAX_ITER = 200
_ROWS = 128
_CHUNK = 128
_TOL = 2e-6


def _stk_kernel(s_ref, o_ref, q0_ref, q2_ref, af0_ref, af2_ref):
    rows = s_ref.shape[0]
    n = s_ref.shape[1]
    n_chunks = n // _CHUNK
    nf = jnp.float32(n)
    nu0n = jnp.float32(_K_TOP)
    nu1n = jnp.float32(n - 2 * _K_TOP)
    nu2n = jnp.float32(_K_TOP)
    one = jnp.float32(1.0)
    tol = jnp.float32(_TOL)

    def fold(p, op):
        t = p[:, 0:128]
        for k in range(1, _CHUNK // 128):
            t = op(t, p[:, k * 128:(k + 1) * 128])
        return t

    add = jnp.add

    first = s_ref[:, 0:_CHUNK]
    mn = fold(first, jnp.minimum)
    mx = fold(first, jnp.maximum)
    for c in range(1, n_chunks):
        sc = s_ref[:, c * _CHUNK:(c + 1) * _CHUNK]
        mn = jnp.minimum(mn, fold(sc, jnp.minimum))
        mx = jnp.maximum(mx, fold(sc, jnp.maximum))
    smin = jnp.min(mn, axis=1, keepdims=True)
    smax = jnp.max(mx, axis=1, keepdims=True)

    g = jnp.float32(1.0 / _EPS) / (smax - smin + jnp.float32(1e-12))
    u0 = jnp.float32(0.25 / _EPS) + smin * g
    u2 = jnp.float32(0.75 / _EPS) + smin * g

    acc1 = jnp.zeros((rows, 128), jnp.float32)
    acc2 = jnp.zeros((rows, 128), jnp.float32)
    for c in range(n_chunks):
        sl = slice(c * _CHUNK, (c + 1) * _CHUNK)
        t = s_ref[:, sl] * g
        q0c = jnp.exp(u0 - t)
        q2c = jnp.exp(t - u2)
        q0_ref[:, sl] = q0c
        q2_ref[:, sl] = q2c
        w = one / (one + q0c + q2c)
        acc1 = acc1 + fold(w, add)
        acc2 = acc2 + fold(q2c * w, add)
    r1 = jnp.sum(acc1, axis=1, keepdims=True)
    r2 = jnp.sum(acc2, axis=1, keepdims=True)

    def column_sums(a0, a1, a2):
        acc1 = jnp.zeros((rows, 128), jnp.float32)
        acc2 = jnp.zeros((rows, 128), jnp.float32)
        for c in range(n_chunks):
            sl = slice(c * _CHUNK, (c + 1) * _CHUNK)
            q0c = q0_ref[:, sl]
            q2c = q2_ref[:, sl]
            w = one / (a1 + q0c * a0 + q2c * a2)
            acc1 = acc1 + fold(w, add)
            acc2 = acc2 + fold(q2c * w, add)
        r1 = jnp.sum(acc1, axis=1, keepdims=True)
        r2 = jnp.sum(acc2, axis=1, keepdims=True)
        return r1, r2

    def new_scales(a0, a1, a2, r1, r2):
        a0n = nu0n * a0 / (nf - a1 * r1 - a2 * r2)
        return a0n, nu1n / r1, nu2n / r2

    a0, a1, a2 = new_scales(one, one, one, r1, r2)

    def cond(carry):
        t, _, _, _, delta = carry
        return jnp.logical_and(t < _MAX_ITER - 1, delta > 0)

    def body(carry):
        t, a0, a1, a2, _ = carry
        r1, r2 = column_sums(a0, a1, a2)
        a0n, a1n, a2n = new_scales(a0, a1, a2, r1, r2)
        d = jnp.maximum(jnp.abs(a0n - a0) - tol * a0,
                        jnp.abs(a1n - a1) - tol * a1)
        d = jnp.maximum(d, jnp.abs(a2n - a2) - tol * a2)
        return (t + 1, a0n, a1n, a2n, jnp.max(d))

    t, a0, a1, a2, delta = jax.lax.while_loop(
        cond, body, (1, a0, a1, a2, jnp.float32(1.0)))

    af0_ref[...] = a0
    af2_ref[...] = a2

    @pl.when(delta > 0)
    def _():
        r1f, r2f = column_sums(a0, a1, a2)
        a0n, _, a2n = new_scales(a0, a1, a2, r1f, r2f)
        af0_ref[...] = a0n
        af2_ref[...] = a2n

    a0n = af0_ref[...]
    a2n = af2_ref[...]
    for c in range(n_chunks):
        sl = slice(c * _CHUNK, (c + 1) * _CHUNK)
        q0c = q0_ref[:, sl]
        q2c = q2_ref[:, sl]
        w = one / (a1 + q0c * a0 + q2c * a2)
        o_ref[:, sl] = (q2c * a2n - q0c * a0n) * w


@jax.jit
def kernel(scores):
    b, n = scores.shape
    return pl.pallas_call(
        _stk_kernel,
        grid=(b // _ROWS,),
        in_specs=[pl.BlockSpec((_ROWS, n), lambda i: (i, 0))],
        out_specs=pl.BlockSpec((_ROWS, n), lambda i: (i, 0)),
        out_shape=jax.ShapeDtypeStruct((b, n), jnp.float32),
        scratch_shapes=[
            pltpu.VMEM((_ROWS, n), jnp.float32),
            pltpu.VMEM((_ROWS, n), jnp.float32),
            pltpu.VMEM((_ROWS, 1), jnp.float32),
            pltpu.VMEM((_ROWS, 1), jnp.float32),
        ],
        compiler_params=pltpu.CompilerParams(
            dimension_semantics=("parallel",),
        ),
    )(scores)


# speedup vs baseline: 1.1466x; 1.0176x over previous
import jax
import jax.numpy as jnp
from jax.experimental import pallas as pl
from jax.experimental.pallas import tpu as pltpu

_K_TOP = 512
_EPS = 0.1
_MAX_ITER = 200
_ROWS = 128
_CHUNK = 256
_TOL = 4e-6


def _stk_kernel(s_ref, o_ref, q0_ref, q2_ref, af0_ref, af2_ref):
    rows = s_ref.shape[0]
    n = s_ref.shape[1]
    n_chunks = n // _CHUNK
    nf = jnp.float32(n)
    nu0n = jnp.float32(_K_TOP)
    nu1n = jnp.float32(n - 2 * _K_TOP)
    nu2n = jnp.float32(_K_TOP)
    one = jnp.float32(1.0)
    tol = jnp.float32(_TOL)

    def fold(p, op):
        t = p[:, 0:128]
        for k in range(1, _CHUNK // 128):
            t = op(t, p[:, k * 128:(k + 1) * 128])
        return t

    add = jnp.add

    first = s_ref[:, 0:_CHUNK]
    mn = fold(first, jnp.minimum)
    mx = fold(first, jnp.maximum)
    for c in range(1, n_chunks):
        sc = s_ref[:, c * _CHUNK:(c + 1) * _CHUNK]
        mn = jnp.minimum(mn, fold(sc, jnp.minimum))
        mx = jnp.maximum(mx, fold(sc, jnp.maximum))
    smin = jnp.min(mn, axis=1, keepdims=True)
    smax = jnp.max(mx, axis=1, keepdims=True)

    g = jnp.float32(1.0 / _EPS) / (smax - smin + jnp.float32(1e-12))
    u0 = jnp.float32(0.25 / _EPS) + smin * g
    u2 = jnp.float32(0.75 / _EPS) + smin * g

    acc1 = jnp.zeros((rows, 128), jnp.float32)
    acc2 = jnp.zeros((rows, 128), jnp.float32)
    for c in range(n_chunks):
        sl = slice(c * _CHUNK, (c + 1) * _CHUNK)
        t = s_ref[:, sl] * g
        q0c = jnp.exp(u0 - t)
        q2c = jnp.exp(t - u2)
        q0_ref[:, sl] = q0c
        q2_ref[:, sl] = q2c
        w = one / (one + q0c + q2c)
        acc1 = acc1 + fold(w, add)
        acc2 = acc2 + fold(q2c * w, add)
    r1 = jnp.sum(acc1, axis=1, keepdims=True)
    r2 = jnp.sum(acc2, axis=1, keepdims=True)

    def column_sums(a0, a1, a2):
        acc1 = jnp.zeros((rows, 128), jnp.float32)
        acc2 = jnp.zeros((rows, 128), jnp.float32)
        for c in range(n_chunks):
            sl = slice(c * _CHUNK, (c + 1) * _CHUNK)
            q0c = q0_ref[:, sl]
            q2c = q2_ref[:, sl]
            w = one / (a1 + q0c * a0 + q2c * a2)
            acc1 = acc1 + fold(w, add)
            acc2 = acc2 + fold(q2c * w, add)
        r1 = jnp.sum(acc1, axis=1, keepdims=True)
        r2 = jnp.sum(acc2, axis=1, keepdims=True)
        return r1, r2

    def new_scales(a0, a1, a2, r1, r2):
        a0n = nu0n * a0 / (nf - a1 * r1 - a2 * r2)
        return a0n, nu1n / r1, nu2n / r2

    a0, a1, a2 = new_scales(one, one, one, r1, r2)

    def cond(carry):
        t, _, _, _, delta = carry
        return jnp.logical_and(t < _MAX_ITER - 1, delta > 0)

    def body(carry):
        t, a0, a1, a2, _ = carry
        r1, r2 = column_sums(a0, a1, a2)
        a0n, a1n, a2n = new_scales(a0, a1, a2, r1, r2)
        d = jnp.maximum(jnp.abs(a0n - a0) - tol * a0,
                        jnp.abs(a1n - a1) - tol * a1)
        d = jnp.maximum(d, jnp.abs(a2n - a2) - tol * a2)
        return (t + 1, a0n, a1n, a2n, jnp.max(d))

    t, a0, a1, a2, delta = jax.lax.while_loop(
        cond, body, (1, a0, a1, a2, jnp.float32(1.0)))

    af0_ref[...] = a0
    af2_ref[...] = a2

    @pl.when(delta > 0)
    def _():
        r1f, r2f = column_sums(a0, a1, a2)
        a0n, _, a2n = new_scales(a0, a1, a2, r1f, r2f)
        af0_ref[...] = a0n
        af2_ref[...] = a2n

    a0n = af0_ref[...]
    a2n = af2_ref[...]
    for c in range(n_chunks):
        sl = slice(c * _CHUNK, (c + 1) * _CHUNK)
        q0c = q0_ref[:, sl]
        q2c = q2_ref[:, sl]
        w = one / (a1 + q0c * a0 + q2c * a2)
        o_ref[:, sl] = (q2c * a2n - q0c * a0n) * w


@jax.jit
def kernel(scores):
    b, n = scores.shape
    return pl.pallas_call(
        _stk_kernel,
        grid=(b // _ROWS,),
        in_specs=[pl.BlockSpec((_ROWS, n), lambda i: (i, 0))],
        out_specs=pl.BlockSpec((_ROWS, n), lambda i: (i, 0)),
        out_shape=jax.ShapeDtypeStruct((b, n), jnp.float32),
        scratch_shapes=[
            pltpu.VMEM((_ROWS, n), jnp.float32),
            pltpu.VMEM((_ROWS, n), jnp.float32),
            pltpu.VMEM((_ROWS, 1), jnp.float32),
            pltpu.VMEM((_ROWS, 1), jnp.float32),
        ],
        compiler_params=pltpu.CompilerParams(
            dimension_semantics=("parallel",),
        ),
    )(scores)


# speedup vs baseline: 1.1471x; 1.0004x over previous
import jax
import jax.numpy as jnp
from jax.experimental import pallas as pl
from jax.experimental.pallas import tpu as pltpu

_K_TOP = 512
_EPS = 0.1
_MAX_ITER = 200
_ROWS = 128
_CHUNK = 256
_TOL = 2e-6


def _stk_kernel(s_ref, o_ref, q0_ref, q2_ref, af0_ref, af2_ref):
    rows = s_ref.shape[0]
    n = s_ref.shape[1]
    n_chunks = n // _CHUNK
    nf = jnp.float32(n)
    nu0n = jnp.float32(_K_TOP)
    nu1n = jnp.float32(n - 2 * _K_TOP)
    nu2n = jnp.float32(_K_TOP)
    one = jnp.float32(1.0)
    tol = jnp.float32(_TOL)

    def fold(p, op):
        t = p[:, 0:128]
        for k in range(1, _CHUNK // 128):
            t = op(t, p[:, k * 128:(k + 1) * 128])
        return t

    add = jnp.add

    first = s_ref[:, 0:_CHUNK]
    mn = fold(first, jnp.minimum)
    mx = fold(first, jnp.maximum)
    for c in range(1, n_chunks):
        sc = s_ref[:, c * _CHUNK:(c + 1) * _CHUNK]
        mn = jnp.minimum(mn, fold(sc, jnp.minimum))
        mx = jnp.maximum(mx, fold(sc, jnp.maximum))
    smin = jnp.min(mn, axis=1, keepdims=True)
    smax = jnp.max(mx, axis=1, keepdims=True)

    g = jnp.float32(1.0 / _EPS) / (smax - smin + jnp.float32(1e-12))
    u0 = jnp.float32(0.25 / _EPS) + smin * g
    u2 = jnp.float32(0.75 / _EPS) + smin * g

    acc1 = jnp.zeros((rows, 128), jnp.float32)
    acc2 = jnp.zeros((rows, 128), jnp.float32)
    for c in range(n_chunks):
        sl = slice(c * _CHUNK, (c + 1) * _CHUNK)
        t = s_ref[:, sl] * g
        q0c = jnp.exp(u0 - t)
        q2c = jnp.exp(t - u2)
        q0_ref[:, sl] = q0c
        q2_ref[:, sl] = q2c
        w = one / (one + q0c + q2c)
        acc1 = acc1 + fold(w, add)
        acc2 = acc2 + fold(q2c * w, add)
    r1 = jnp.sum(acc1, axis=1, keepdims=True)
    r2 = jnp.sum(acc2, axis=1, keepdims=True)

    def column_sums(a0, a1, a2):
        acc1 = jnp.zeros((rows, 128), jnp.float32)
        acc2 = jnp.zeros((rows, 128), jnp.float32)
        for c in range(n_chunks):
            sl = slice(c * _CHUNK, (c + 1) * _CHUNK)
            q0c = q0_ref[:, sl]
            q2c = q2_ref[:, sl]
            w = one / (a1 + q0c * a0 + q2c * a2)
            acc1 = acc1 + fold(w, add)
            acc2 = acc2 + fold(q2c * w, add)
        r1 = jnp.sum(acc1, axis=1, keepdims=True)
        r2 = jnp.sum(acc2, axis=1, keepdims=True)
        return r1, r2

    def new_scales(a0, a1, a2, r1, r2):
        a0n = nu0n * a0 / (nf - a1 * r1 - a2 * r2)
        return a0n, nu1n / r1, nu2n / r2

    a0, a1, a2 = new_scales(one, one, one, r1, r2)

    def cond(carry):
        t, _, _, _, delta = carry
        return jnp.logical_and(t < _MAX_ITER - 1, delta > 0)

    def body(carry):
        t, a0, a1, a2, _ = carry
        r1, r2 = column_sums(a0, a1, a2)
        a0n, a1n, a2n = new_scales(a0, a1, a2, r1, r2)
        d = jnp.maximum(jnp.abs(a0n - a0) - tol * a0,
                        jnp.abs(a1n - a1) - tol * a1)
        d = jnp.maximum(d, jnp.abs(a2n - a2) - tol * a2)
        return (t + 1, a0n, a1n, a2n, jnp.max(d))

    t, a0, a1, a2, delta = jax.lax.while_loop(
        cond, body, (1, a0, a1, a2, jnp.float32(1.0)))

    af0_ref[...] = a0
    af2_ref[...] = a2

    @pl.when(delta > 0)
    def _():
        r1f, r2f = column_sums(a0, a1, a2)
        a0n, _, a2n = new_scales(a0, a1, a2, r1f, r2f)
        af0_ref[...] = a0n
        af2_ref[...] = a2n

    a0n = af0_ref[...]
    a2n = af2_ref[...]
    for c in range(n_chunks):
        sl = slice(c * _CHUNK, (c + 1) * _CHUNK)
        q0c = q0_ref[:, sl]
        q2c = q2_ref[:, sl]
        w = one / (a1 + q0c * a0 + q2c * a2)
        o_ref[:, sl] = (q2c * a2n - q0c * a0n) * w


@jax.jit
def kernel(scores):
    b, n = scores.shape
    return pl.pallas_call(
        _stk_kernel,
        grid=(b // _ROWS,),
        in_specs=[pl.BlockSpec((_ROWS, n), lambda i: (i, 0))],
        out_specs=pl.BlockSpec((_ROWS, n), lambda i: (i, 0)),
        out_shape=jax.ShapeDtypeStruct((b, n), jnp.float32),
        scratch_shapes=[
            pltpu.VMEM((_ROWS, n), jnp.float32),
            pltpu.VMEM((_ROWS, n), jnp.float32),
            pltpu.VMEM((_ROWS, 1), jnp.float32),
            pltpu.VMEM((_ROWS, 1), jnp.float32),
        ],
        compiler_params=pltpu.CompilerParams(
            dimension_semantics=("parallel",),
            flags={"XLA_TPU_STORE_TO_LOAD_FORWARDING_WINDOW": 12288},
        ),
    )(scores)
